# baseline (speedup 1.0000x reference)
import functools

import jax
import jax.numpy as jnp
from jax.experimental import pallas as pl
from jax.experimental.pallas import tpu as pltpu

MAX_LEVELS = 3
_KCHUNK = 2048
_KCHUNK_B = 10000
_PRECISION = jax.lax.Precision.DEFAULT


def _pick(n, cands):
    for c in cands:
        if n % c == 0:
            return c
    return n


def _kchunks(n, step=None):
    step = _KCHUNK if step is None else step
    offs = []
    o = 0
    while o < n:
        sz = min(step, n - o)
        offs.append((o, sz))
        o += sz
    return tuple(offs)


def _conv_update(n2npool, conv_w_ref, conv_b_ref, in_lin_rows, old_rows):
    node_linear = jnp.dot(n2npool, conv_w_ref[...], precision=_PRECISION)
    merged = node_linear + conv_b_ref[...] + in_lin_rows
    return jnp.maximum(merged, 0.0) + old_rows


def _pool(adj_bf_ref, ebf_ref, base, n_nodes):
    acc = None
    for off, sz in _kchunks(n_nodes):
        part = jnp.dot(adj_bf_ref[:, off:off + sz],
                       ebf_ref[pl.ds(base + off, sz), :],
                       precision=_PRECISION,
                       preferred_element_type=jnp.float32)
        acc = part if acc is None else acc + part
    return acc


def _body_a(adj_ref, x_ref, w_n2l_ref, b_n2l_ref, conv_w_ref, conv_b_ref,
            in_lin_ref, embed1_ref, embed1bf_ref, adj_bf_ref,
            embed0_ref,
            *, bm, rb, n_nodes):
    i = pl.program_id(0)

    @pl.when(i == 0)
    def _init():
        def chunk(r, carry):
            xa = x_ref[pl.ds(r * rb, rb), :]
            il = jnp.dot(xa, w_n2l_ref[...], precision=_PRECISION)
            il = il + b_n2l_ref[...]
            in_lin_ref[pl.ds(r * rb, rb), :] = il
            embed0_ref[pl.ds(r * rb, rb), :] = jnp.maximum(il, 0.0)
            return carry
        jax.lax.fori_loop(0, n_nodes // rb, chunk, 0)

    n2npool = None
    for off, sz in _kchunks(n_nodes):
        a_blk = adj_ref[:, off:off + sz]
        adj_bf_ref[:, off:off + sz] = a_blk.astype(jnp.bfloat16)
        part = jnp.dot(a_blk, embed0_ref[pl.ds(off, sz), :],
                       precision=_PRECISION)
        n2npool = part if n2npool is None else n2npool + part

    in_lin_rows = in_lin_ref[pl.ds(i * bm, bm), :]
    old_rows = embed0_ref[pl.ds(i * bm, bm), :]
    new_embed = _conv_update(n2npool, conv_w_ref, conv_b_ref,
                             in_lin_rows, old_rows)
    embed1_ref[...] = new_embed
    embed1bf_ref[...] = new_embed.astype(jnp.bfloat16)


def _body_b(adj_bf_ref, embed1bf_win, embed1_win, in_lin_ref, labels_ref,
            conv_w_ref, conv_b_ref, last_w_ref, last_b_ref,
            pred_ref, loss_ref, acc_ref,
            embed_ref, lsum_ref,
            *, bm, n_nodes, n_classes, n_blocks):
    lv = pl.program_id(0)
    i = pl.program_id(1)
    in_lin_rows = in_lin_ref[...]

    @pl.when(lv == 0)
    def _level1():
        n2npool = None
        for off, sz in _kchunks(n_nodes, _KCHUNK_B):
            part = jnp.dot(adj_bf_ref[:, off:off + sz],
                           embed1bf_win[pl.ds(off, sz), :],
                           precision=_PRECISION,
                           preferred_element_type=jnp.float32)
            n2npool = part if n2npool is None else n2npool + part
        old_rows = embed1_win[pl.ds(i * bm, bm), :]
        new_embed = _conv_update(n2npool, conv_w_ref, conv_b_ref,
                                 in_lin_rows, old_rows)
        embed_ref[pl.ds(i * bm, bm), :] = new_embed

    @pl.when(lv == 1)
    def _level2_head():
        n2npool = None
        for off, sz in _kchunks(n_nodes, _KCHUNK_B):
            e_chunk = embed_ref[pl.ds(off, sz), :].astype(jnp.bfloat16)
            part = jnp.dot(adj_bf_ref[:, off:off + sz], e_chunk,
                           precision=_PRECISION,
                           preferred_element_type=jnp.float32)
            n2npool = part if n2npool is None else n2npool + part
        old_rows = embed_ref[pl.ds(i * bm, bm), :]
        new_embed = _conv_update(n2npool, conv_w_ref, conv_b_ref,
                                 in_lin_rows, old_rows)

        logits = jnp.dot(new_embed, last_w_ref[...], precision=_PRECISION)
        logits = logits + last_b_ref[...]
        m = jnp.max(logits, axis=1, keepdims=True)
        shifted = logits - m
        lse = jnp.log(jnp.sum(jnp.exp(shifted), axis=1, keepdims=True))
        ls = shifted - lse
        col = jax.lax.broadcasted_iota(jnp.int32, ls.shape, 1)
        is_max = logits >= m
        pred = jnp.min(jnp.where(is_max, col, n_classes), axis=1)
        pred_ref[0, 0, :] = pred
        labels = labels_ref[0, 0, :]
        acc_ref[0, 0, :] = (pred == labels).astype(jnp.int32)
        sel = jnp.sum(jnp.where(col == labels[:, None], ls, 0.0), axis=1)
        part_loss = jnp.sum(sel)
        total = jnp.where(i == 0, part_loss, lsum_ref[0] + part_loss)
        lsum_ref[0] = total

        @pl.when(i == n_blocks - 1)
        def _emit_loss():
            loss_ref[...] = jnp.full((1, 1), -total / n_nodes, jnp.float32)


def kernel(x, adj, labels, w_n2l, b_n2l, conv_w, conv_b, last_w, last_b):
    n, f_dim = x.shape
    l_dim = conv_w.shape[0]
    n_classes = last_w.shape[1]
    bma = _pick(n, (200, 256, 128, 100, 80, 50, 40, 25, 16, 10, 8, 5, 4, 2, 1))
    bmb = _pick(n, (1000, 1024, 512, 256, 200, 128, 100, 80, 50, 40, 25, 16,
                    10, 8, 5, 4, 2, 1))
    rb = _pick(n, (1000, 800, 500, 400, 200, 100, 50, 25, 10, 8, 5, 4, 2, 1))
    nia = n // bma
    nib = n // bmb

    labels3 = labels.astype(jnp.int32).reshape(nib, 1, bmb)
    b_n2l2 = b_n2l.reshape(1, l_dim)
    conv_b2 = conv_b.reshape(1, l_dim)
    last_b2 = last_b.reshape(1, n_classes)

    body_a = functools.partial(_body_a, bm=bma, rb=rb, n_nodes=n)
    in_lin, embed1, embed1bf, adj_bf = pl.pallas_call(
        body_a,
        grid=(nia,),
        in_specs=[
            pl.BlockSpec((bma, n), lambda i: (i, 0)),
            pl.BlockSpec((n, f_dim), lambda i: (0, 0)),
            pl.BlockSpec((f_dim, l_dim), lambda i: (0, 0)),
            pl.BlockSpec((1, l_dim), lambda i: (0, 0)),
            pl.BlockSpec((l_dim, l_dim), lambda i: (0, 0)),
            pl.BlockSpec((1, l_dim), lambda i: (0, 0)),
        ],
        out_specs=(
            pl.BlockSpec((n, l_dim), lambda i: (0, 0)),
            pl.BlockSpec((bma, l_dim), lambda i: (i, 0)),
            pl.BlockSpec((bma, l_dim), lambda i: (i, 0)),
            pl.BlockSpec((bma, n), lambda i: (i, 0)),
        ),
        out_shape=(
            jax.ShapeDtypeStruct((n, l_dim), jnp.float32),
            jax.ShapeDtypeStruct((n, l_dim), jnp.float32),
            jax.ShapeDtypeStruct((n, l_dim), jnp.bfloat16),
            jax.ShapeDtypeStruct((n, n), jnp.bfloat16),
        ),
        scratch_shapes=[
            pltpu.VMEM((n, l_dim), jnp.float32),
        ],
        compiler_params=pltpu.CompilerParams(
            dimension_semantics=("arbitrary",),
        ),
    )(adj, x, w_n2l, b_n2l2, conv_w, conv_b2)

    body_b = functools.partial(_body_b, bm=bmb, n_nodes=n,
                               n_classes=n_classes, n_blocks=nib)
    pred3, loss2, acc3 = pl.pallas_call(
        body_b,
        grid=(MAX_LEVELS - 1, nib),
        in_specs=[
            pl.BlockSpec((bmb, n), lambda lv, i: (i, 0)),
            pl.BlockSpec((n, l_dim), lambda lv, i: (0, 0)),
            pl.BlockSpec((n, l_dim), lambda lv, i: (0, 0)),
            pl.BlockSpec((bmb, l_dim), lambda lv, i: (i, 0)),
            pl.BlockSpec((1, 1, bmb), lambda lv, i: (i, 0, 0)),
            pl.BlockSpec((l_dim, l_dim), lambda lv, i: (0, 0)),
            pl.BlockSpec((1, l_dim), lambda lv, i: (0, 0)),
            pl.BlockSpec((l_dim, n_classes), lambda lv, i: (0, 0)),
            pl.BlockSpec((1, n_classes), lambda lv, i: (0, 0)),
        ],
        out_specs=(
            pl.BlockSpec((1, 1, bmb), lambda lv, i: (i, 0, 0)),
            pl.BlockSpec((1, 1), lambda lv, i: (0, 0)),
            pl.BlockSpec((1, 1, bmb), lambda lv, i: (i, 0, 0)),
        ),
        out_shape=(
            jax.ShapeDtypeStruct((nib, 1, bmb), jnp.int32),
            jax.ShapeDtypeStruct((1, 1), jnp.float32),
            jax.ShapeDtypeStruct((nib, 1, bmb), jnp.int32),
        ),
        scratch_shapes=[
            pltpu.VMEM((n, l_dim), jnp.float32),
            pltpu.SMEM((1,), jnp.float32),
        ],
        compiler_params=pltpu.CompilerParams(
            dimension_semantics=("arbitrary", "arbitrary"),
        ),
    )(adj_bf, embed1bf, embed1, in_lin, labels3, conv_w, conv_b2, last_w,
      last_b2)

    pred = pred3.reshape(n, 1)
    loss = loss2[0, 0]
    acc = acc3.reshape(n) != 0
    return pred, loss, acc


# speedup vs baseline: 1.0025x; 1.0025x over previous
import functools

import jax
import jax.numpy as jnp
from jax.experimental import pallas as pl
from jax.experimental.pallas import tpu as pltpu

MAX_LEVELS = 3
_KCHUNK = 2048
_KCHUNK_B = 10000
_PRECISION = jax.lax.Precision.DEFAULT


def _pick(n, cands):
    for c in cands:
        if n % c == 0:
            return c
    return n


def _kchunks(n, step=None):
    step = _KCHUNK if step is None else step
    offs = []
    o = 0
    while o < n:
        sz = min(step, n - o)
        offs.append((o, sz))
        o += sz
    return tuple(offs)


def _conv_update(n2npool, conv_w_ref, conv_b_ref, in_lin_rows, old_rows):
    node_linear = jnp.dot(n2npool, conv_w_ref[...], precision=_PRECISION)
    merged = node_linear + conv_b_ref[...] + in_lin_rows
    return jnp.maximum(merged, 0.0) + old_rows


def _body_a(adj_ref, x_ref, w_n2l_ref, b_n2l_ref, conv_w_ref, conv_b_ref,
            in_lin_ref, embed1_ref, embed1bf_ref, adj_bf_ref,
            embed0_ref,
            *, bm, rb, n_nodes):
    i = pl.program_id(0)

    @pl.when(i == 0)
    def _init():
        def chunk(r, carry):
            xa = x_ref[pl.ds(r * rb, rb), :]
            il = jnp.dot(xa, w_n2l_ref[...], precision=_PRECISION)
            il = il + b_n2l_ref[...]
            in_lin_ref[pl.ds(r * rb, rb), :] = il
            embed0_ref[pl.ds(r * rb, rb), :] = jnp.maximum(il, 0.0)
            return carry
        jax.lax.fori_loop(0, n_nodes // rb, chunk, 0)

    n2npool = None
    for off, sz in _kchunks(n_nodes):
        a_blk = adj_ref[:, off:off + sz]
        adj_bf_ref[:, off:off + sz] = a_blk.astype(jnp.bfloat16)
        part = jnp.dot(a_blk, embed0_ref[pl.ds(off, sz), :],
                       precision=_PRECISION)
        n2npool = part if n2npool is None else n2npool + part

    in_lin_rows = in_lin_ref[pl.ds(i * bm, bm), :]
    old_rows = embed0_ref[pl.ds(i * bm, bm), :]
    new_embed = _conv_update(n2npool, conv_w_ref, conv_b_ref,
                             in_lin_rows, old_rows)
    embed1_ref[...] = new_embed
    embed1bf_ref[...] = new_embed.astype(jnp.bfloat16)


def _body_b(adj_bf_ref, embed1bf_win, embed1_win, in_lin_ref, labels_ref,
            conv_w_ref, conv_b_ref, last_w_ref, last_b_ref,
            pred_ref, loss_ref, acc_ref,
            embed_ref, lsum_ref,
            *, bm, n_nodes, n_classes, n_blocks):
    lv = pl.program_id(0)
    i = pl.program_id(1)
    in_lin_rows = in_lin_ref[...]

    @pl.when(lv == 0)
    def _level1():
        n2npool = None
        for off, sz in _kchunks(n_nodes, _KCHUNK_B):
            part = jnp.dot(adj_bf_ref[:, off:off + sz],
                           embed1bf_win[pl.ds(off, sz), :],
                           precision=_PRECISION,
                           preferred_element_type=jnp.float32)
            n2npool = part if n2npool is None else n2npool + part
        old_rows = embed1_win[pl.ds(i * bm, bm), :]
        new_embed = _conv_update(n2npool, conv_w_ref, conv_b_ref,
                                 in_lin_rows, old_rows)
        embed_ref[pl.ds(i * bm, bm), :] = new_embed

    @pl.when(lv == 1)
    def _level2_head():
        n2npool = None
        for off, sz in _kchunks(n_nodes, _KCHUNK_B):
            e_chunk = embed_ref[pl.ds(off, sz), :].astype(jnp.bfloat16)
            part = jnp.dot(adj_bf_ref[:, off:off + sz], e_chunk,
                           precision=_PRECISION,
                           preferred_element_type=jnp.float32)
            n2npool = part if n2npool is None else n2npool + part
        old_rows = embed_ref[pl.ds(i * bm, bm), :]
        new_embed = _conv_update(n2npool, conv_w_ref, conv_b_ref,
                                 in_lin_rows, old_rows)

        logits = jnp.dot(new_embed, last_w_ref[...], precision=_PRECISION)
        logits = logits + last_b_ref[...]
        m = jnp.max(logits, axis=1, keepdims=True)
        shifted = logits - m
        lse = jnp.log(jnp.sum(jnp.exp(shifted), axis=1, keepdims=True))
        ls = shifted - lse
        col = jax.lax.broadcasted_iota(jnp.int32, ls.shape, 1)
        is_max = logits >= m
        pred = jnp.min(jnp.where(is_max, col, n_classes), axis=1)
        pred_ref[0, 0, :] = pred
        labels = labels_ref[0, 0, :]
        acc_ref[0, 0, :] = (pred == labels).astype(jnp.int32)
        sel = jnp.sum(jnp.where(col == labels[:, None], ls, 0.0), axis=1)
        part_loss = jnp.sum(sel)
        total = jnp.where(i == 0, part_loss, lsum_ref[0] + part_loss)
        lsum_ref[0] = total

        @pl.when(i == n_blocks - 1)
        def _emit_loss():
            loss_ref[...] = jnp.full((1, 1), -total / n_nodes, jnp.float32)


def kernel(x, adj, labels, w_n2l, b_n2l, conv_w, conv_b, last_w, last_b):
    n, f_dim = x.shape
    l_dim = conv_w.shape[0]
    n_classes = last_w.shape[1]
    bma = _pick(n, (200, 256, 128, 100, 80, 50, 40, 25, 16, 10, 8, 5, 4, 2, 1))
    bmb = _pick(n, (1000, 1024, 512, 256, 200, 128, 100, 80, 50, 40, 25, 16,
                    10, 8, 5, 4, 2, 1))
    rb = _pick(n, (1000, 800, 500, 400, 200, 100, 50, 25, 10, 8, 5, 4, 2, 1))
    nia = n // bma
    nib = n // bmb

    labels3 = labels.astype(jnp.int32).reshape(nib, 1, bmb)
    b_n2l2 = b_n2l.reshape(1, l_dim)
    conv_b2 = conv_b.reshape(1, l_dim)
    last_b2 = last_b.reshape(1, n_classes)

    body_a = functools.partial(_body_a, bm=bma, rb=rb, n_nodes=n)
    in_lin, embed1, embed1bf, adj_bf = pl.pallas_call(
        body_a,
        grid=(nia,),
        in_specs=[
            pl.BlockSpec((bma, n), lambda i: (i, 0)),
            pl.BlockSpec((n, f_dim), lambda i: (0, 0)),
            pl.BlockSpec((f_dim, l_dim), lambda i: (0, 0)),
            pl.BlockSpec((1, l_dim), lambda i: (0, 0)),
            pl.BlockSpec((l_dim, l_dim), lambda i: (0, 0)),
            pl.BlockSpec((1, l_dim), lambda i: (0, 0)),
        ],
        out_specs=(
            pl.BlockSpec((n, l_dim), lambda i: (0, 0)),
            pl.BlockSpec((bma, l_dim), lambda i: (i, 0)),
            pl.BlockSpec((bma, l_dim), lambda i: (i, 0)),
            pl.BlockSpec((bma, n), lambda i: (i, 0)),
        ),
        out_shape=(
            jax.ShapeDtypeStruct((n, l_dim), jnp.float32),
            jax.ShapeDtypeStruct((n, l_dim), jnp.float32),
            jax.ShapeDtypeStruct((n, l_dim), jnp.bfloat16),
            jax.ShapeDtypeStruct((n, n), jnp.bfloat16),
        ),
        scratch_shapes=[
            pltpu.VMEM((n, l_dim), jnp.float32),
        ],
        compiler_params=pltpu.CompilerParams(
            dimension_semantics=("arbitrary",),
        ),
    )(adj, x, w_n2l, b_n2l2, conv_w, conv_b2)

    body_b = functools.partial(_body_b, bm=bmb, n_nodes=n,
                               n_classes=n_classes, n_blocks=nib)
    pred3, loss2, acc3 = pl.pallas_call(
        body_b,
        grid=(MAX_LEVELS - 1, nib),
        in_specs=[
            pl.BlockSpec((bmb, n), lambda lv, i: (i, 0)),
            pl.BlockSpec((n, l_dim), lambda lv, i: (0, 0)),
            pl.BlockSpec((n, l_dim), lambda lv, i: (0, 0)),
            pl.BlockSpec((bmb, l_dim), lambda lv, i: (i, 0)),
            pl.BlockSpec((1, 1, bmb), lambda lv, i: (i, 0, 0)),
            pl.BlockSpec((l_dim, l_dim), lambda lv, i: (0, 0)),
            pl.BlockSpec((1, l_dim), lambda lv, i: (0, 0)),
            pl.BlockSpec((l_dim, n_classes), lambda lv, i: (0, 0)),
            pl.BlockSpec((1, n_classes), lambda lv, i: (0, 0)),
        ],
        out_specs=(
            pl.BlockSpec((1, 1, bmb), lambda lv, i: (i, 0, 0)),
            pl.BlockSpec((1, 1), lambda lv, i: (0, 0)),
            pl.BlockSpec((1, 1, bmb), lambda lv, i: (i, 0, 0)),
        ),
        out_shape=(
            jax.ShapeDtypeStruct((nib, 1, bmb), jnp.int32),
            jax.ShapeDtypeStruct((1, 1), jnp.float32),
            jax.ShapeDtypeStruct((nib, 1, bmb), jnp.int32),
        ),
        scratch_shapes=[
            pltpu.VMEM((n, l_dim), jnp.float32),
            pltpu.SMEM((1,), jnp.float32),
        ],
        compiler_params=pltpu.CompilerParams(
            dimension_semantics=("arbitrary", "arbitrary"),
        ),
    )(adj_bf, embed1bf, embed1, in_lin, labels3, conv_w, conv_b2, last_w,
      last_b2)

    pred = pred3.reshape(n, 1)
    loss = loss2[0, 0]
    acc = acc3.reshape(n) != 0
    return pred, loss, acc
